# baseline (speedup 1.0000x reference)
import functools

import jax
import jax.numpy as jnp
from jax import lax
from jax.experimental import pallas as pl
from jax.experimental.pallas import tpu as pltpu
from jax.experimental.pallas import tpu_sc as plsc

_NC = 2
_NS = 16
_NW = _NC * _NS
_L = 16
_CHUNK = 128
_PW = 128
_DUMMY = 16384


def _extract_body(d, npan, ppt, ct_t, tail, se, slots, sbt, ce,
                  se_v, or_v, st_v, pans, rows, didxs,
                  semp, sems_a, sems_b):
  wid = lax.axis_index("s") * _NC + lax.axis_index("c")
  lo = wid * ppt
  hi = jnp.minimum(lo + ppt, npan)

  pltpu.sync_copy(se, se_v)
  pltpu.sync_copy(slots, or_v)
  pltpu.sync_copy(sbt.at[wid], st_v)

  iota = lax.iota(jnp.int32, _L)

  def fire_panel(p, buf):
    @pl.when(jnp.logical_and(p < hi, p < npan - 1))
    def _():
      pltpu.async_copy(ct_t.at[:, pl.ds(p * _PW, _PW)], buf, semp)

    @pl.when(jnp.logical_and(p < hi, p == npan - 1))
    def _():
      pltpu.async_copy(tail, buf, semp)

  def wait_panel(p, buf):
    del p
    pltpu.make_async_copy(ct_t.at[:, pl.ds(0, _PW)], buf, semp).wait()

  for k in range(3):
    fire_panel(lo + k, pans[k])

  def panel_groups(g, _):
    sta = st_v[pl.ds(g * _L, _L)]
    stb = st_v[pl.ds(g * _L + _L, _L)]
    for lane in range(_L):
      p = lo + g * _L + lane
      buf = pans[lane % 4]
      rowbuf = rows[lane % 2]
      didx = didxs[lane % 2]
      sems = sems_a if lane % 2 == 0 else sems_b
      fire_panel(p + 3, pans[(lane + 3) % 4])

      @pl.when(p < hi)
      def _process():
        wait_panel(p, buf)
        s = sta[lane]
        e = stb[0] if lane == _L - 1 else sta[lane + 1]
        cnt = e - s
        tmax = jnp.maximum(1, (cnt + _L - 1) // _L)

        def hit_group(t, carry):
          @pl.when(jnp.logical_or(t > 0, p - lo >= 2))
          def _():
            pltpu.make_async_copy(ce.at[pl.ds(0, _L)], rowbuf, sems).wait()

          off = s + t * _L
          rem = cnt - t * _L
          msk = iota < rem
          ent16 = se_v[pl.ds(off, _L)]
          slot16 = or_v[pl.ds(off, _L)]
          col = lax.bitwise_and(ent16, _PW - 1)
          for f in range(d):
            fv = jnp.full((_L,), f, jnp.int32)
            v = plsc.load_gather(buf, [fv, col], mask=msk)
            plsc.store_scatter(rowbuf, [iota, fv], v, mask=msk)
          didx[...] = jnp.where(msk, slot16, _DUMMY + iota)
          pltpu.async_copy(rowbuf, ce.at[didx], sems)
          return carry

        lax.fori_loop(0, tmax, hit_group, 0)

    return _

  ngrp = (hi - lo + _L - 1) // _L
  lax.fori_loop(0, ngrp, panel_groups, 0)

  pltpu.make_async_copy(ce.at[pl.ds(0, _L)], rows[0], sems_a).wait()
  pltpu.make_async_copy(ce.at[pl.ds(0, _L)], rows[1], sems_b).wait()


def _sc_extract(ct_t, tail, se, slots, sbt, npan, ppt):
  d = ct_t.shape[0]
  mesh = plsc.VectorSubcoreMesh(
      core_axis_name="c", subcore_axis_name="s",
      num_cores=_NC, num_subcores=_NS)
  f = pl.kernel(
      functools.partial(_extract_body, d, npan, ppt),
      out_type=jax.ShapeDtypeStruct((_DUMMY + _L, 2 * d), jnp.float32),
      mesh=mesh,
      compiler_params=pltpu.CompilerParams(
          needs_layout_passes=False, use_tc_tiling_on_sc=True),
      scratch_types=[
          pltpu.VMEM((se.shape[0],), jnp.int32),
          pltpu.VMEM((slots.shape[0],), jnp.int32),
          pltpu.VMEM((sbt.shape[1],), jnp.int32),
          [pltpu.VMEM((d, _PW), jnp.float32) for _ in range(4)],
          [pltpu.VMEM((_L, 2 * d), jnp.float32) for _ in range(2)],
          [pltpu.VMEM((_L,), jnp.int32) for _ in range(2)],
          pltpu.SemaphoreType.DMA,
          pltpu.SemaphoreType.DMA,
          pltpu.SemaphoreType.DMA,
      ],
  )
  return f(ct_t, tail, se, slots, sbt)


def _mul_body(nch, d, ce, ptable2, hidx, tidx, pidx, atom_out,
              hidx_v, tidx_v, pidx_v, pq_v, pp_v,
              hrow, trow, ppair, prod, sem):
  rpw = nch * _CHUNK
  wid = lax.axis_index("s") * _NC + lax.axis_index("c")
  base = wid * rpw

  pltpu.sync_copy(hidx.at[wid], hidx_v)
  pltpu.sync_copy(tidx.at[wid], tidx_v)
  pltpu.sync_copy(pidx.at[wid], pidx_v)

  for j in range(nch):
    for i in range(_CHUNK // _L):
      s = pl.ds(i * _L, _L)
      pid = pidx_v[j, s]
      pq_v[j, s] = lax.shift_right_logical(pid, 1)
      pp_v[j, s] = lax.bitwise_and(pid, 1)

  for j in range(nch):
    cp_h = pltpu.async_copy(ce.at[hidx_v.at[j]], hrow, sem)
    cp_t = pltpu.async_copy(ce.at[tidx_v.at[j]], trow, sem)
    cp_p = pltpu.async_copy(ptable2.at[pq_v.at[j]], ppair, sem)
    cp_h.wait()
    cp_t.wait()
    cp_p.wait()

    def group_body(g, carry):
      gs = pl.ds(g * _L, _L)
      pp16 = pp_v[j, gs]
      for lane in range(_L):
        r = g * _L + lane
        po = pp16[lane] * d
        for c in range(d // _L):
          co = c * _L
          prod[r, pl.ds(co, _L)] = (
              hrow[r, pl.ds(co, _L)]
              * trow[r, pl.ds(co, _L)]
              * ppair[r, pl.ds(po + co, _L)])
      return carry

    lax.fori_loop(0, _CHUNK // _L, group_body, 0)

    pltpu.sync_copy(prod, atom_out.at[pl.ds(base + j * _CHUNK, _CHUNK)])


def _sc_gather_mul(ce, ptable2, hidx, tidx, pidx, n, d):
  nch = (n // _NW) // _CHUNK
  mesh = plsc.VectorSubcoreMesh(
      core_axis_name="c", subcore_axis_name="s",
      num_cores=_NC, num_subcores=_NS)
  f = pl.kernel(
      functools.partial(_mul_body, nch, d),
      out_type=jax.ShapeDtypeStruct((n, d), jnp.float32),
      mesh=mesh,
      compiler_params=pltpu.CompilerParams(
          needs_layout_passes=False, use_tc_tiling_on_sc=True),
      scratch_types=[
          pltpu.VMEM((nch, _CHUNK), jnp.int32),
          pltpu.VMEM((nch, _CHUNK), jnp.int32),
          pltpu.VMEM((nch, _CHUNK), jnp.int32),
          pltpu.VMEM((nch, _CHUNK), jnp.int32),
          pltpu.VMEM((nch, _CHUNK), jnp.int32),
          pltpu.VMEM((_CHUNK, 2 * d), jnp.float32),
          pltpu.VMEM((_CHUNK, 2 * d), jnp.float32),
          pltpu.VMEM((_CHUNK, 2 * d), jnp.float32),
          pltpu.VMEM((_CHUNK, d), jnp.float32),
          pltpu.SemaphoreType.DMA,
      ],
  )
  return f(ce, ptable2, hidx, tidx, pidx)


def _mm_body(atom_ref, w_ref, b_ref, o_ref):
  o_ref[...] = (
      jnp.dot(atom_ref[...], w_ref[...], preferred_element_type=jnp.float32)
      + b_ref[...])


def _out_proj(atom, w, b):
  n, d = atom.shape
  bm = 2048
  return pl.pallas_call(
      _mm_body,
      grid=(n // bm,),
      in_specs=[
          pl.BlockSpec((bm, d), lambda i: (i, 0)),
          pl.BlockSpec((d, d), lambda i: (0, 0)),
          pl.BlockSpec((1, d), lambda i: (0, 0)),
      ],
      out_specs=pl.BlockSpec((bm, d), lambda i: (i, 0)),
      out_shape=jax.ShapeDtypeStruct((n, d), jnp.float32),
  )(atom, w, b.reshape(1, d))


def kernel(constant_table, predicate_table, W_out, b_out, x_entity,
           pred_ids, const_idx):
  n = pred_ids.shape[0]
  m = x_entity.shape[0]
  v, d = constant_table.shape
  nch = (n // _NW) // _CHUNK
  npan = (v + _PW - 1) // _PW
  ppt = (npan + _NW - 1) // _NW

  ct_t = constant_table.T
  tail = jnp.pad(ct_t[:, (npan - 1) * _PW:],
                 ((0, 0), (0, npan * _PW - v)))
  ptable2 = predicate_table.reshape(predicate_table.shape[0] // 2, 2 * d)

  xe = x_entity.astype(jnp.int32)
  se = jnp.sort(xe)
  slots = jnp.argsort(xe).astype(jnp.int32)
  bounds = (jnp.arange(npan + 1, dtype=jnp.int32) * _PW).astype(jnp.int32)
  starts = jnp.searchsorted(se, bounds).astype(jnp.int32)
  row_idx = jnp.minimum(
      jnp.arange(256, dtype=jnp.int32)[None, :]
      + (jnp.arange(_NW, dtype=jnp.int32) * ppt)[:, None], npan)
  sbt = starts[row_idx]
  se = jnp.pad(se, (0, _L))
  slots = jnp.pad(slots, (0, _L))

  hidx = const_idx[:, 0].astype(jnp.int32).reshape(_NW, nch, _CHUNK)
  tidx = const_idx[:, 1].astype(jnp.int32).reshape(_NW, nch, _CHUNK)
  pidx = pred_ids.astype(jnp.int32).reshape(_NW, nch, _CHUNK)

  ce = _sc_extract(ct_t, tail, se, slots, sbt, npan, ppt)
  atom = _sc_gather_mul(ce, ptable2, hidx, tidx, pidx, n, d)
  return _out_proj(atom, W_out, b_out)


# speedup vs baseline: 1.0045x; 1.0045x over previous
import functools

import jax
import jax.numpy as jnp
from jax import lax
from jax.experimental import pallas as pl
from jax.experimental.pallas import tpu as pltpu
from jax.experimental.pallas import tpu_sc as plsc

_NC = 2
_NS = 16
_NW = _NC * _NS
_L = 16
_CHUNK = 128
_PW = 128
_DUMMY = 16384


def _extract_body(d, npan, ppt, ct_t, tail, se, slots, sbt, ce,
                  se_v, or_v, st_v, pans, rows, didxs, semp, semss):
  wid = lax.axis_index("s") * _NC + lax.axis_index("c")
  lo = wid * ppt
  hi = jnp.minimum(lo + ppt, npan)

  pltpu.sync_copy(se, se_v)
  pltpu.sync_copy(slots, or_v)
  pltpu.sync_copy(sbt.at[wid], st_v)

  iota = lax.iota(jnp.int32, _L)

  def fire_panel(p, buf):
    @pl.when(jnp.logical_and(p < hi, p < npan - 1))
    def _():
      pltpu.async_copy(ct_t.at[:, pl.ds(p * _PW, _PW)], buf, semp)

    @pl.when(jnp.logical_and(p < hi, p == npan - 1))
    def _():
      pltpu.async_copy(tail, buf, semp)

  def wait_panel(p, buf):
    del p
    pltpu.make_async_copy(ct_t.at[:, pl.ds(0, _PW)], buf, semp).wait()

  for k in range(7):
    fire_panel(lo + k, pans[k])

  def panel_groups(g, _):
    sta = st_v[pl.ds(g * _L, _L)]
    stb = st_v[pl.ds(g * _L + _L, _L)]
    for lane in range(_L):
      p = lo + g * _L + lane
      buf = pans[lane % 8]
      rowbuf = rows[lane % 8]
      didx = didxs[lane % 8]
      sems = semss[lane % 8]
      fire_panel(p + 7, pans[(lane + 7) % 8])

      @pl.when(p < hi)
      def _process():
        wait_panel(p, buf)
        s = sta[lane]
        e = stb[0] if lane == _L - 1 else sta[lane + 1]
        cnt = e - s
        tmax = jnp.maximum(1, (cnt + _L - 1) // _L)

        def hit_group(t, carry):
          @pl.when(jnp.logical_or(t > 0, p - lo >= 8))
          def _():
            pltpu.make_async_copy(ce.at[pl.ds(0, _L)], rowbuf, sems).wait()

          off = s + t * _L
          rem = cnt - t * _L
          msk = iota < rem
          ent16 = se_v[pl.ds(off, _L)]
          slot16 = or_v[pl.ds(off, _L)]
          col = lax.bitwise_and(ent16, _PW - 1)
          for f in range(d):
            fv = jnp.full((_L,), f, jnp.int32)
            v = plsc.load_gather(buf, [fv, col], mask=msk)
            plsc.store_scatter(rowbuf, [iota, fv], v, mask=msk)
          didx[...] = jnp.where(msk, slot16, _DUMMY + iota)
          pltpu.async_copy(rowbuf, ce.at[didx], sems)
          return carry

        lax.fori_loop(0, tmax, hit_group, 0)

    return _

  ngrp = (hi - lo + _L - 1) // _L
  lax.fori_loop(0, ngrp, panel_groups, 0)

  for k in range(8):
    pltpu.make_async_copy(ce.at[pl.ds(0, _L)], rows[k], semss[k]).wait()


def _sc_extract(ct_t, tail, se, slots, sbt, npan, ppt):
  d = ct_t.shape[0]
  mesh = plsc.VectorSubcoreMesh(
      core_axis_name="c", subcore_axis_name="s",
      num_cores=_NC, num_subcores=_NS)
  f = pl.kernel(
      functools.partial(_extract_body, d, npan, ppt),
      out_type=jax.ShapeDtypeStruct((_DUMMY + _L, 2 * d), jnp.float32),
      mesh=mesh,
      compiler_params=pltpu.CompilerParams(
          needs_layout_passes=False, use_tc_tiling_on_sc=True),
      scratch_types=[
          pltpu.VMEM((se.shape[0],), jnp.int32),
          pltpu.VMEM((slots.shape[0],), jnp.int32),
          pltpu.VMEM((sbt.shape[1],), jnp.int32),
          [pltpu.VMEM((d, _PW), jnp.float32) for _ in range(8)],
          [pltpu.VMEM((_L, 2 * d), jnp.float32) for _ in range(8)],
          [pltpu.VMEM((_L,), jnp.int32) for _ in range(8)],
          pltpu.SemaphoreType.DMA,
          [pltpu.SemaphoreType.DMA for _ in range(8)],
      ],
  )
  return f(ct_t, tail, se, slots, sbt)


def _mul_body(nch, d, ce, ptable2, hidx, tidx, pidx, atom_out,
              hidx_v, tidx_v, pidx_v, pq_v, pp_v,
              hrow, trow, ppair, prod, sem):
  rpw = nch * _CHUNK
  wid = lax.axis_index("s") * _NC + lax.axis_index("c")
  base = wid * rpw

  pltpu.sync_copy(hidx.at[wid], hidx_v)
  pltpu.sync_copy(tidx.at[wid], tidx_v)
  pltpu.sync_copy(pidx.at[wid], pidx_v)

  for j in range(nch):
    for i in range(_CHUNK // _L):
      s = pl.ds(i * _L, _L)
      pid = pidx_v[j, s]
      pq_v[j, s] = lax.shift_right_logical(pid, 1)
      pp_v[j, s] = lax.bitwise_and(pid, 1)

  for j in range(nch):
    cp_h = pltpu.async_copy(ce.at[hidx_v.at[j]], hrow, sem)
    cp_t = pltpu.async_copy(ce.at[tidx_v.at[j]], trow, sem)
    cp_p = pltpu.async_copy(ptable2.at[pq_v.at[j]], ppair, sem)
    cp_h.wait()
    cp_t.wait()
    cp_p.wait()

    def group_body(g, carry):
      gs = pl.ds(g * _L, _L)
      pp16 = pp_v[j, gs]
      for lane in range(_L):
        r = g * _L + lane
        po = pp16[lane] * d
        for c in range(d // _L):
          co = c * _L
          prod[r, pl.ds(co, _L)] = (
              hrow[r, pl.ds(co, _L)]
              * trow[r, pl.ds(co, _L)]
              * ppair[r, pl.ds(po + co, _L)])
      return carry

    lax.fori_loop(0, _CHUNK // _L, group_body, 0)

    pltpu.sync_copy(prod, atom_out.at[pl.ds(base + j * _CHUNK, _CHUNK)])


def _sc_gather_mul(ce, ptable2, hidx, tidx, pidx, n, d):
  nch = (n // _NW) // _CHUNK
  mesh = plsc.VectorSubcoreMesh(
      core_axis_name="c", subcore_axis_name="s",
      num_cores=_NC, num_subcores=_NS)
  f = pl.kernel(
      functools.partial(_mul_body, nch, d),
      out_type=jax.ShapeDtypeStruct((n, d), jnp.float32),
      mesh=mesh,
      compiler_params=pltpu.CompilerParams(
          needs_layout_passes=False, use_tc_tiling_on_sc=True),
      scratch_types=[
          pltpu.VMEM((nch, _CHUNK), jnp.int32),
          pltpu.VMEM((nch, _CHUNK), jnp.int32),
          pltpu.VMEM((nch, _CHUNK), jnp.int32),
          pltpu.VMEM((nch, _CHUNK), jnp.int32),
          pltpu.VMEM((nch, _CHUNK), jnp.int32),
          pltpu.VMEM((_CHUNK, 2 * d), jnp.float32),
          pltpu.VMEM((_CHUNK, 2 * d), jnp.float32),
          pltpu.VMEM((_CHUNK, 2 * d), jnp.float32),
          pltpu.VMEM((_CHUNK, d), jnp.float32),
          pltpu.SemaphoreType.DMA,
      ],
  )
  return f(ce, ptable2, hidx, tidx, pidx)


def _mm_body(atom_ref, w_ref, b_ref, o_ref):
  o_ref[...] = (
      jnp.dot(atom_ref[...], w_ref[...], preferred_element_type=jnp.float32)
      + b_ref[...])


def _out_proj(atom, w, b):
  n, d = atom.shape
  bm = 2048
  return pl.pallas_call(
      _mm_body,
      grid=(n // bm,),
      in_specs=[
          pl.BlockSpec((bm, d), lambda i: (i, 0)),
          pl.BlockSpec((d, d), lambda i: (0, 0)),
          pl.BlockSpec((1, d), lambda i: (0, 0)),
      ],
      out_specs=pl.BlockSpec((bm, d), lambda i: (i, 0)),
      out_shape=jax.ShapeDtypeStruct((n, d), jnp.float32),
  )(atom, w, b.reshape(1, d))


def kernel(constant_table, predicate_table, W_out, b_out, x_entity,
           pred_ids, const_idx):
  n = pred_ids.shape[0]
  m = x_entity.shape[0]
  v, d = constant_table.shape
  nch = (n // _NW) // _CHUNK
  npan = (v + _PW - 1) // _PW
  ppt = (npan + _NW - 1) // _NW

  ct_t = constant_table.T
  tail = jnp.pad(ct_t[:, (npan - 1) * _PW:],
                 ((0, 0), (0, npan * _PW - v)))
  ptable2 = predicate_table.reshape(predicate_table.shape[0] // 2, 2 * d)

  xe = x_entity.astype(jnp.int32)
  se = jnp.sort(xe)
  slots = jnp.argsort(xe).astype(jnp.int32)
  bounds = (jnp.arange(npan + 1, dtype=jnp.int32) * _PW).astype(jnp.int32)
  starts = jnp.searchsorted(se, bounds).astype(jnp.int32)
  row_idx = jnp.minimum(
      jnp.arange(256, dtype=jnp.int32)[None, :]
      + (jnp.arange(_NW, dtype=jnp.int32) * ppt)[:, None], npan)
  sbt = starts[row_idx]
  se = jnp.pad(se, (0, _L))
  slots = jnp.pad(slots, (0, _L))

  hidx = const_idx[:, 0].astype(jnp.int32).reshape(_NW, nch, _CHUNK)
  tidx = const_idx[:, 1].astype(jnp.int32).reshape(_NW, nch, _CHUNK)
  pidx = pred_ids.astype(jnp.int32).reshape(_NW, nch, _CHUNK)

  ce = _sc_extract(ct_t, tail, se, slots, sbt, npan, ppt)
  atom = _sc_gather_mul(ce, ptable2, hidx, tidx, pidx, n, d)
  return _out_proj(atom, W_out, b_out)


# speedup vs baseline: 1.9924x; 1.9835x over previous
import functools

import jax
import jax.numpy as jnp
from jax import lax
from jax.experimental import pallas as pl
from jax.experimental.pallas import tpu as pltpu
from jax.experimental.pallas import tpu_sc as plsc

_NC = 2
_NS = 16
_NW = _NC * _NS
_L = 16
_CHUNK = 128
_PW = 512
_DUMMY = 16384


def _extract_body(d, npan, ppt, ct_t, tail, se, slots, sbt, ce,
                  se_v, or_v, st_v, pans, rows, didxs, semp, semss):
  wid = lax.axis_index("s") * _NC + lax.axis_index("c")
  lo = wid * ppt
  hi = jnp.minimum(lo + ppt, npan)

  pltpu.sync_copy(se, se_v)
  pltpu.sync_copy(slots, or_v)
  pltpu.sync_copy(sbt.at[wid], st_v)

  iota = lax.iota(jnp.int32, _L)

  def fire_panel(p, buf):
    @pl.when(jnp.logical_and(p < hi, p < npan - 1))
    def _():
      pltpu.async_copy(ct_t.at[:, pl.ds(p * _PW, _PW)], buf, semp)

    @pl.when(jnp.logical_and(p < hi, p == npan - 1))
    def _():
      pltpu.async_copy(tail, buf, semp)

  def wait_panel(p, buf):
    del p
    pltpu.make_async_copy(ct_t.at[:, pl.ds(0, _PW)], buf, semp).wait()

  fire_panel(lo, pans[0])

  def panel_groups(g, _):
    sta = st_v[pl.ds(g * _L, _L)]
    stb = st_v[pl.ds(g * _L + _L, _L)]
    for lane in range(_L):
      p = lo + g * _L + lane
      buf = pans[lane % 2]
      rowbuf = rows[lane % 8]
      didx = didxs[lane % 8]
      sems = semss[lane % 8]
      fire_panel(p + 1, pans[(lane + 1) % 2])

      @pl.when(p < hi)
      def _process():
        wait_panel(p, buf)
        s = sta[lane]
        e = stb[0] if lane == _L - 1 else sta[lane + 1]
        cnt = e - s
        tmax = jnp.maximum(1, (cnt + _L - 1) // _L)

        def hit_group(t, carry):
          @pl.when(jnp.logical_or(t > 0, p - lo >= 8))
          def _():
            pltpu.make_async_copy(ce.at[pl.ds(0, _L)], rowbuf, sems).wait()

          off = s + t * _L
          rem = cnt - t * _L
          msk = iota < rem
          ent16 = se_v[pl.ds(off, _L)]
          slot16 = or_v[pl.ds(off, _L)]
          col = lax.bitwise_and(ent16, _PW - 1)
          for f in range(d):
            fv = jnp.full((_L,), f, jnp.int32)
            v = plsc.load_gather(buf, [fv, col], mask=msk)
            plsc.store_scatter(rowbuf, [iota, fv], v, mask=msk)
          didx[...] = jnp.where(msk, slot16, _DUMMY + iota)
          pltpu.async_copy(rowbuf, ce.at[didx], sems)
          return carry

        lax.fori_loop(0, tmax, hit_group, 0)

    return _

  ngrp = (hi - lo + _L - 1) // _L
  lax.fori_loop(0, ngrp, panel_groups, 0)

  for k in range(8):
    pltpu.make_async_copy(ce.at[pl.ds(0, _L)], rows[k], semss[k]).wait()


def _sc_extract(ct_t, tail, se, slots, sbt, npan, ppt):
  d = ct_t.shape[0]
  mesh = plsc.VectorSubcoreMesh(
      core_axis_name="c", subcore_axis_name="s",
      num_cores=_NC, num_subcores=_NS)
  f = pl.kernel(
      functools.partial(_extract_body, d, npan, ppt),
      out_type=jax.ShapeDtypeStruct((_DUMMY + _L, 2 * d), jnp.float32),
      mesh=mesh,
      compiler_params=pltpu.CompilerParams(
          needs_layout_passes=False, use_tc_tiling_on_sc=True),
      scratch_types=[
          pltpu.VMEM((se.shape[0],), jnp.int32),
          pltpu.VMEM((slots.shape[0],), jnp.int32),
          pltpu.VMEM((sbt.shape[1],), jnp.int32),
          [pltpu.VMEM((d, _PW), jnp.float32) for _ in range(2)],
          [pltpu.VMEM((_L, 2 * d), jnp.float32) for _ in range(8)],
          [pltpu.VMEM((_L,), jnp.int32) for _ in range(8)],
          pltpu.SemaphoreType.DMA,
          [pltpu.SemaphoreType.DMA for _ in range(8)],
      ],
  )
  return f(ct_t, tail, se, slots, sbt)


def _mul_body(nch, d, ce, ptable2, hidx, tidx, pidx, atom_out,
              hidx_v, tidx_v, pidx_v, pq_v, pp_v,
              hrow, trow, ppair, prod, sem):
  rpw = nch * _CHUNK
  wid = lax.axis_index("s") * _NC + lax.axis_index("c")
  base = wid * rpw

  pltpu.sync_copy(hidx.at[wid], hidx_v)
  pltpu.sync_copy(tidx.at[wid], tidx_v)
  pltpu.sync_copy(pidx.at[wid], pidx_v)

  for j in range(nch):
    for i in range(_CHUNK // _L):
      s = pl.ds(i * _L, _L)
      pid = pidx_v[j, s]
      pq_v[j, s] = lax.shift_right_logical(pid, 1)
      pp_v[j, s] = lax.bitwise_and(pid, 1)

  for j in range(nch):
    cp_h = pltpu.async_copy(ce.at[hidx_v.at[j]], hrow, sem)
    cp_t = pltpu.async_copy(ce.at[tidx_v.at[j]], trow, sem)
    cp_p = pltpu.async_copy(ptable2.at[pq_v.at[j]], ppair, sem)
    cp_h.wait()
    cp_t.wait()
    cp_p.wait()

    def group_body(g, carry):
      gs = pl.ds(g * _L, _L)
      pp16 = pp_v[j, gs]
      for lane in range(_L):
        r = g * _L + lane
        po = pp16[lane] * d
        for c in range(d // _L):
          co = c * _L
          prod[r, pl.ds(co, _L)] = (
              hrow[r, pl.ds(co, _L)]
              * trow[r, pl.ds(co, _L)]
              * ppair[r, pl.ds(po + co, _L)])
      return carry

    lax.fori_loop(0, _CHUNK // _L, group_body, 0)

    pltpu.sync_copy(prod, atom_out.at[pl.ds(base + j * _CHUNK, _CHUNK)])


def _sc_gather_mul(ce, ptable2, hidx, tidx, pidx, n, d):
  nch = (n // _NW) // _CHUNK
  mesh = plsc.VectorSubcoreMesh(
      core_axis_name="c", subcore_axis_name="s",
      num_cores=_NC, num_subcores=_NS)
  f = pl.kernel(
      functools.partial(_mul_body, nch, d),
      out_type=jax.ShapeDtypeStruct((n, d), jnp.float32),
      mesh=mesh,
      compiler_params=pltpu.CompilerParams(
          needs_layout_passes=False, use_tc_tiling_on_sc=True),
      scratch_types=[
          pltpu.VMEM((nch, _CHUNK), jnp.int32),
          pltpu.VMEM((nch, _CHUNK), jnp.int32),
          pltpu.VMEM((nch, _CHUNK), jnp.int32),
          pltpu.VMEM((nch, _CHUNK), jnp.int32),
          pltpu.VMEM((nch, _CHUNK), jnp.int32),
          pltpu.VMEM((_CHUNK, 2 * d), jnp.float32),
          pltpu.VMEM((_CHUNK, 2 * d), jnp.float32),
          pltpu.VMEM((_CHUNK, 2 * d), jnp.float32),
          pltpu.VMEM((_CHUNK, d), jnp.float32),
          pltpu.SemaphoreType.DMA,
      ],
  )
  return f(ce, ptable2, hidx, tidx, pidx)


def _mm_body(atom_ref, w_ref, b_ref, o_ref):
  o_ref[...] = (
      jnp.dot(atom_ref[...], w_ref[...], preferred_element_type=jnp.float32)
      + b_ref[...])


def _out_proj(atom, w, b):
  n, d = atom.shape
  bm = 2048
  return pl.pallas_call(
      _mm_body,
      grid=(n // bm,),
      in_specs=[
          pl.BlockSpec((bm, d), lambda i: (i, 0)),
          pl.BlockSpec((d, d), lambda i: (0, 0)),
          pl.BlockSpec((1, d), lambda i: (0, 0)),
      ],
      out_specs=pl.BlockSpec((bm, d), lambda i: (i, 0)),
      out_shape=jax.ShapeDtypeStruct((n, d), jnp.float32),
  )(atom, w, b.reshape(1, d))


def kernel(constant_table, predicate_table, W_out, b_out, x_entity,
           pred_ids, const_idx):
  n = pred_ids.shape[0]
  m = x_entity.shape[0]
  v, d = constant_table.shape
  nch = (n // _NW) // _CHUNK
  npan = (v + _PW - 1) // _PW
  ppt = (npan + _NW - 1) // _NW

  ct_t = constant_table.T
  tail = jnp.pad(ct_t[:, (npan - 1) * _PW:],
                 ((0, 0), (0, npan * _PW - v)))
  ptable2 = predicate_table.reshape(predicate_table.shape[0] // 2, 2 * d)

  xe = x_entity.astype(jnp.int32)
  se = jnp.sort(xe)
  slots = jnp.argsort(xe).astype(jnp.int32)
  bounds = (jnp.arange(npan + 1, dtype=jnp.int32) * _PW).astype(jnp.int32)
  starts = jnp.searchsorted(se, bounds).astype(jnp.int32)
  row_idx = jnp.minimum(
      jnp.arange(256, dtype=jnp.int32)[None, :]
      + (jnp.arange(_NW, dtype=jnp.int32) * ppt)[:, None], npan)
  sbt = starts[row_idx]
  se = jnp.pad(se, (0, _L))
  slots = jnp.pad(slots, (0, _L))

  hidx = const_idx[:, 0].astype(jnp.int32).reshape(_NW, nch, _CHUNK)
  tidx = const_idx[:, 1].astype(jnp.int32).reshape(_NW, nch, _CHUNK)
  pidx = pred_ids.astype(jnp.int32).reshape(_NW, nch, _CHUNK)

  ce = _sc_extract(ct_t, tail, se, slots, sbt, npan, ppt)
  atom = _sc_gather_mul(ce, ptable2, hidx, tidx, pidx, n, d)
  return _out_proj(atom, W_out, b_out)


# speedup vs baseline: 2.9897x; 1.5005x over previous
import functools

import jax
import jax.numpy as jnp
from jax import lax
from jax.experimental import pallas as pl
from jax.experimental.pallas import tpu as pltpu
from jax.experimental.pallas import tpu_sc as plsc

_NC = 2
_NS = 16
_NW = _NC * _NS
_L = 16
_CHUNK = 128
_PW = 512
_DUMMY = 16384


def _extract_body(d, npan, ppt, ct_t, tail, se, slots, sbt, ce,
                  se_v, or_v, st_v, pans, rows, didxs, semp, semss):
  wid = lax.axis_index("s") * _NC + lax.axis_index("c")
  lo = wid * ppt
  hi = jnp.minimum(lo + ppt, npan)

  pltpu.sync_copy(se, se_v)
  pltpu.sync_copy(slots, or_v)
  pltpu.sync_copy(sbt.at[wid], st_v)

  iota = lax.iota(jnp.int32, _L)

  def fire_panel(p, buf):
    @pl.when(jnp.logical_and(p < hi, p < npan - 1))
    def _():
      pltpu.async_copy(ct_t.at[:, pl.ds(p * _PW, _PW)], buf, semp)

    @pl.when(jnp.logical_and(p < hi, p == npan - 1))
    def _():
      pltpu.async_copy(tail, buf, semp)

  def wait_panel(p, buf):
    del p
    pltpu.make_async_copy(ct_t.at[:, pl.ds(0, _PW)], buf, semp).wait()

  fire_panel(lo, pans[0])

  def panel_groups(g, _):
    sta = st_v[pl.ds(g * _L, _L)]
    stb = st_v[pl.ds(g * _L + _L, _L)]
    for lane in range(_L):
      p = lo + g * _L + lane
      buf = pans[lane % 2]
      rowbuf = rows[lane % 8]
      didx = didxs[lane % 8]
      sems = semss[lane % 8]
      fire_panel(p + 1, pans[(lane + 1) % 2])

      @pl.when(p < hi)
      def _process():
        wait_panel(p, buf)
        s = sta[lane]
        e = stb[0] if lane == _L - 1 else sta[lane + 1]
        cnt = e - s
        tmax = jnp.maximum(1, (cnt + _L - 1) // _L)

        def hit_group(t, carry):
          @pl.when(jnp.logical_or(t > 0, p - lo >= 8))
          def _():
            pltpu.make_async_copy(ce.at[pl.ds(0, _L)], rowbuf, sems).wait()

          off = s + t * _L
          rem = cnt - t * _L
          msk = iota < rem
          ent16 = se_v[pl.ds(off, _L)]
          slot16 = or_v[pl.ds(off, _L)]
          col = lax.bitwise_and(ent16, _PW - 1)
          for f in range(d):
            fv = jnp.full((_L,), f, jnp.int32)
            v = plsc.load_gather(buf, [fv, col], mask=msk)
            plsc.store_scatter(rowbuf, [iota, fv], v, mask=msk)
          didx[...] = jnp.where(msk, slot16, _DUMMY + iota)
          pltpu.async_copy(rowbuf, ce.at[didx], sems)
          return carry

        lax.fori_loop(0, tmax, hit_group, 0)

    return _

  ngrp = (hi - lo + _L - 1) // _L
  lax.fori_loop(0, ngrp, panel_groups, 0)

  for k in range(8):
    pltpu.make_async_copy(ce.at[pl.ds(0, _L)], rows[k], semss[k]).wait()


def _sc_extract(ct_t, tail, se, slots, sbt, npan, ppt):
  d = ct_t.shape[0]
  mesh = plsc.VectorSubcoreMesh(
      core_axis_name="c", subcore_axis_name="s",
      num_cores=_NC, num_subcores=_NS)
  f = pl.kernel(
      functools.partial(_extract_body, d, npan, ppt),
      out_type=jax.ShapeDtypeStruct((_DUMMY + _L, 2 * d), jnp.float32),
      mesh=mesh,
      compiler_params=pltpu.CompilerParams(
          needs_layout_passes=False, use_tc_tiling_on_sc=True),
      scratch_types=[
          pltpu.VMEM((se.shape[0],), jnp.int32),
          pltpu.VMEM((slots.shape[0],), jnp.int32),
          pltpu.VMEM((sbt.shape[1],), jnp.int32),
          [pltpu.VMEM((d, _PW), jnp.float32) for _ in range(2)],
          [pltpu.VMEM((_L, 2 * d), jnp.float32) for _ in range(8)],
          [pltpu.VMEM((_L,), jnp.int32) for _ in range(8)],
          pltpu.SemaphoreType.DMA,
          [pltpu.SemaphoreType.DMA for _ in range(8)],
      ],
  )
  return f(ct_t, tail, se, slots, sbt)


def _mul_body(nch, d, ce, ptable2, hidx, tidx, pidx, atom_out,
              hidx_v, tidx_v, pidx_v, pq_v, pp_v,
              hrow, trow, ppair, prod, sem):
  rpw = nch * _CHUNK
  wid = lax.axis_index("s") * _NC + lax.axis_index("c")
  base = wid * rpw

  pltpu.sync_copy(hidx.at[wid], hidx_v)
  pltpu.sync_copy(tidx.at[wid], tidx_v)
  pltpu.sync_copy(pidx.at[wid], pidx_v)

  for j in range(nch):
    for i in range(_CHUNK // _L):
      s = pl.ds(i * _L, _L)
      pid = pidx_v[j, s]
      pq_v[j, s] = lax.shift_right_logical(pid, 1)
      pp_v[j, s] = lax.bitwise_and(pid, 1)

  for j in range(nch):
    cp_h = pltpu.async_copy(ce.at[hidx_v.at[j]], hrow, sem)
    cp_t = pltpu.async_copy(ce.at[tidx_v.at[j]], trow, sem)
    cp_p = pltpu.async_copy(ptable2.at[pq_v.at[j]], ppair, sem)
    cp_h.wait()
    cp_t.wait()
    cp_p.wait()

    def group_body(g, carry):
      gs = pl.ds(g * _L, _L)
      pp16 = pp_v[j, gs]
      for lane in range(_L):
        r = g * _L + lane
        po = pp16[lane] * d
        for c in range(d // _L):
          co = c * _L
          prod[r, pl.ds(co, _L)] = (
              hrow[r, pl.ds(co, _L)]
              * trow[r, pl.ds(co, _L)]
              * ppair[r, pl.ds(po + co, _L)])
      return carry

    lax.fori_loop(0, _CHUNK // _L, group_body, 0)

    pltpu.sync_copy(prod, atom_out.at[pl.ds(base + j * _CHUNK, _CHUNK)])


def _sc_gather_mul(ce, ptable2, hidx, tidx, pidx, n, d):
  nch = (n // _NW) // _CHUNK
  mesh = plsc.VectorSubcoreMesh(
      core_axis_name="c", subcore_axis_name="s",
      num_cores=_NC, num_subcores=_NS)
  f = pl.kernel(
      functools.partial(_mul_body, nch, d),
      out_type=jax.ShapeDtypeStruct((n, d), jnp.float32),
      mesh=mesh,
      compiler_params=pltpu.CompilerParams(
          needs_layout_passes=False, use_tc_tiling_on_sc=True),
      scratch_types=[
          pltpu.VMEM((nch, _CHUNK), jnp.int32),
          pltpu.VMEM((nch, _CHUNK), jnp.int32),
          pltpu.VMEM((nch, _CHUNK), jnp.int32),
          pltpu.VMEM((nch, _CHUNK), jnp.int32),
          pltpu.VMEM((nch, _CHUNK), jnp.int32),
          pltpu.VMEM((_CHUNK, 2 * d), jnp.float32),
          pltpu.VMEM((_CHUNK, 2 * d), jnp.float32),
          pltpu.VMEM((_CHUNK, 2 * d), jnp.float32),
          pltpu.VMEM((_CHUNK, d), jnp.float32),
          pltpu.SemaphoreType.DMA,
      ],
  )
  return f(ce, ptable2, hidx, tidx, pidx)


def _mm_body(atom_ref, w_ref, b_ref, o_ref):
  o_ref[...] = (
      jnp.dot(atom_ref[...], w_ref[...], preferred_element_type=jnp.float32)
      + b_ref[...])


def _out_proj(atom, w, b):
  n, d = atom.shape
  bm = 2048
  return pl.pallas_call(
      _mm_body,
      grid=(n // bm,),
      in_specs=[
          pl.BlockSpec((bm, d), lambda i: (i, 0)),
          pl.BlockSpec((d, d), lambda i: (0, 0)),
          pl.BlockSpec((1, d), lambda i: (0, 0)),
      ],
      out_specs=pl.BlockSpec((bm, d), lambda i: (i, 0)),
      out_shape=jax.ShapeDtypeStruct((n, d), jnp.float32),
  )(atom, w, b.reshape(1, d))


def kernel(constant_table, predicate_table, W_out, b_out, x_entity,
           pred_ids, const_idx):
  n = pred_ids.shape[0]
  m = x_entity.shape[0]
  v, d = constant_table.shape
  nch = (n // _NW) // _CHUNK
  npan = (v + _PW - 1) // _PW
  ppt = (npan + _NW - 1) // _NW

  ct_t = constant_table.T
  tail = jnp.pad(ct_t[:, (npan - 1) * _PW:],
                 ((0, 0), (0, npan * _PW - v)))
  ptable2 = predicate_table.reshape(predicate_table.shape[0] // 2, 2 * d)

  xe = x_entity.astype(jnp.int32)
  se, slots = lax.sort((xe, jnp.arange(m, dtype=jnp.int32)), num_keys=1)
  bounds = (jnp.arange(npan + 1, dtype=jnp.int32) * _PW).astype(jnp.int32)
  starts = jnp.searchsorted(se, bounds,
                            method="compare_all").astype(jnp.int32)
  row_idx = jnp.minimum(
      jnp.arange(256, dtype=jnp.int32)[None, :]
      + (jnp.arange(_NW, dtype=jnp.int32) * ppt)[:, None], npan)
  sbt = starts[row_idx]
  se = jnp.pad(se, (0, _L))
  slots = jnp.pad(slots, (0, _L))

  hidx = const_idx[:, 0].astype(jnp.int32).reshape(_NW, nch, _CHUNK)
  tidx = const_idx[:, 1].astype(jnp.int32).reshape(_NW, nch, _CHUNK)
  pidx = pred_ids.astype(jnp.int32).reshape(_NW, nch, _CHUNK)

  ce = _sc_extract(ct_t, tail, se, slots, sbt, npan, ppt)
  atom = _sc_gather_mul(ce, ptable2, hidx, tidx, pidx, n, d)
  return _out_proj(atom, W_out, b_out)
